# Initial kernel scaffold; baseline (speedup 1.0000x reference)
import jax
import jax.numpy as jnp
from jax.experimental import pallas as pl


def kernel(z, codebook):
    raise NotImplementedError("write your pallas kernel here")


# speedup vs baseline: 1.0982x; 1.0982x over previous
import jax
import jax.numpy as jnp
from jax.experimental import pallas as pl
from jax.experimental.pallas import tpu as pltpu

_SCALES = [1, 2, 4, 8, 16, 32, 64]
_K = 8192
_D = 256
_BM = 256


def _interp(x, size):
    S_in = x.shape[1]
    if S_in == size:
        return x
    pos = (jnp.arange(size, dtype=jnp.float32) + 0.5) * (S_in / size) - 0.5
    pos = jnp.clip(pos, 0.0, float(S_in - 1))
    lo = jnp.floor(pos).astype(jnp.int32)
    hi = jnp.minimum(lo + 1, S_in - 1)
    w = (pos - lo.astype(jnp.float32))[None, :, None]
    return jnp.take(x, lo, axis=1) * (1.0 - w) + jnp.take(x, hi, axis=1) * w


def _nn_body(r_ref, r2_ref, cb_ref, c2_ref, out_ref):
    dot = jax.lax.dot_general(r_ref[...], cb_ref[...],
                              (((1,), (1,)), ((), ())),
                              preferred_element_type=jnp.float32)
    d = (r2_ref[...] - 2.0 * dot) + c2_ref[...]
    m = jnp.min(d, axis=1, keepdims=True)
    ii = jax.lax.broadcasted_iota(jnp.int32, d.shape, 1)
    am = jnp.min(jnp.where(d == m, ii, _K), axis=1, keepdims=True)
    out_ref[...] = am


def _nn_idx(r, codebook, c2):
    N = r.shape[0]
    r2 = jnp.sum(r * r, axis=-1, keepdims=True)
    out = pl.pallas_call(
        _nn_body,
        grid=(N // _BM,),
        in_specs=[
            pl.BlockSpec((_BM, _D), lambda i: (i, 0)),
            pl.BlockSpec((_BM, 1), lambda i: (i, 0)),
            pl.BlockSpec((_K, _D), lambda i: (0, 0)),
            pl.BlockSpec((1, _K), lambda i: (0, 0)),
        ],
        out_specs=pl.BlockSpec((_BM, 1), lambda i: (i, 0)),
        out_shape=jax.ShapeDtypeStruct((N, 1), jnp.int32),
        compiler_params=pltpu.CompilerParams(
            dimension_semantics=("arbitrary",)),
    )(r, r2, codebook, c2.reshape(1, _K))
    return out.reshape(N)


def kernel(z, codebook):
    B, S_last, D = z.shape
    c2 = jnp.sum(codebook * codebook, axis=-1)
    f_hat = jnp.zeros_like(z)
    for s in _SCALES:
        r = _interp(z - f_hat, s)
        idx = _nn_idx(r.reshape(B * s, D), codebook, c2)
        q = jnp.take(codebook, idx.reshape(B, s), axis=0)
        f_hat = f_hat + _interp(q, S_last)
    commitment_loss = jnp.mean((jax.lax.stop_gradient(f_hat) - z) ** 2)
    q_latent_loss = jnp.mean((f_hat - jax.lax.stop_gradient(z)) ** 2)
    return f_hat, commitment_loss, q_latent_loss


# speedup vs baseline: 1.4983x; 1.3644x over previous
import functools

import numpy as np
import jax
import jax.numpy as jnp
from jax import lax
from jax.experimental import pallas as pl
from jax.experimental.pallas import tpu as pltpu
from jax.experimental.pallas import tpu_sc as plsc

_SCALES = [1, 2, 4, 8, 16, 32, 64]
_K = 8192
_D = 256
_BM = 256
_BB = 8


def _interp_wts(S_in, size):
    pos = (np.arange(size, dtype=np.float32) + np.float32(0.5)) \
        * np.float32(S_in / size) - np.float32(0.5)
    pos = np.clip(pos, np.float32(0.0), np.float32(S_in - 1))
    lo = np.floor(pos).astype(np.int32)
    hi = np.minimum(lo + 1, S_in - 1).astype(np.int32)
    w = (pos - lo.astype(np.float32)).astype(np.float32)
    return lo, hi, w


def _nn_body(r_ref, cb_ref, c2_ref, out_ref):
    r = r_ref[...]
    r2 = jnp.sum(r * r, axis=1, keepdims=True)
    dot = lax.dot_general(r, cb_ref[...], (((1,), (1,)), ((), ())),
                          preferred_element_type=jnp.float32)
    d = (r2 - 2.0 * dot) + c2_ref[...]
    m = jnp.min(d, axis=1, keepdims=True)
    ii = lax.broadcasted_iota(jnp.int32, d.shape, 1)
    am = jnp.min(jnp.where(d == m, ii, _K), axis=1, keepdims=True)
    out_ref[...] = am


def _nn_idx(r_flat, codebook, c2):
    N = r_flat.shape[0]
    out = pl.pallas_call(
        _nn_body,
        grid=(N // _BM,),
        in_specs=[
            pl.BlockSpec((_BM, _D), lambda i: (i, 0)),
            pl.BlockSpec((_K, _D), lambda i: (0, 0)),
            pl.BlockSpec((1, _K), lambda i: (0, 0)),
        ],
        out_specs=pl.BlockSpec((_BM, 1), lambda i: (i, 0)),
        out_shape=jax.ShapeDtypeStruct((N, 1), jnp.int32),
        compiler_params=pltpu.CompilerParams(
            dimension_semantics=("arbitrary",)),
    )(r_flat, codebook, c2)
    return out.reshape(N)


def _down0_body(s, down, z_ref, r_ref):
    lo, hi, w = down
    for i in range(s):
        wi = np.float32(w[i])
        wi1 = np.float32(1.0) - wi
        r_ref[i, :, :] = z_ref[:, int(lo[i]), :] * wi1 \
            + z_ref[:, int(hi[i]), :] * wi


def _down_first(z, s):
    B = z.shape[0]
    body = functools.partial(_down0_body, s, _interp_wts(64, s))
    return pl.pallas_call(
        body,
        grid=(B // _BB,),
        in_specs=[pl.BlockSpec((_BB, 64, _D), lambda b: (b, 0, 0))],
        out_specs=pl.BlockSpec((s, _BB, _D), lambda b: (0, b, 0)),
        out_shape=jax.ShapeDtypeStruct((s, B, _D), jnp.float32),
    )(z)


def _glue_body(s_prev, s, up, down, has_f, z_ref, *refs):
    if has_f:
        f_ref, q_ref, fout_ref, r_ref = refs
    else:
        q_ref, fout_ref, r_ref = refs
        f_ref = None
    lo_u, hi_u, w_u = up
    lo_d, hi_d, w_d = down
    for t in range(64):
        if s_prev == 64:
            y = q_ref[t, :, :]
        else:
            wt = np.float32(w_u[t])
            wt1 = np.float32(1.0) - wt
            y = q_ref[int(lo_u[t]), :, :] * wt1 + q_ref[int(hi_u[t]), :, :] * wt
        fout_ref[:, t, :] = y if f_ref is None else f_ref[:, t, :] + y
    for i in range(s):
        if s == 64:
            r_ref[i, :, :] = z_ref[:, i, :] - fout_ref[:, i, :]
        else:
            wi = np.float32(w_d[i])
            wi1 = np.float32(1.0) - wi
            ra = z_ref[:, int(lo_d[i]), :] - fout_ref[:, int(lo_d[i]), :]
            rb = z_ref[:, int(hi_d[i]), :] - fout_ref[:, int(hi_d[i]), :]
            r_ref[i, :, :] = ra * wi1 + rb * wi


def _glue(z, f_hat, q, s_prev, s):
    B = z.shape[0]
    up = _interp_wts(s_prev, 64)
    down = _interp_wts(64, s)
    has_f = f_hat is not None
    body = functools.partial(_glue_body, s_prev, s, up, down, has_f)
    in_specs = [pl.BlockSpec((_BB, 64, _D), lambda b: (b, 0, 0))]
    args = [z]
    if has_f:
        in_specs.append(pl.BlockSpec((_BB, 64, _D), lambda b: (b, 0, 0)))
        args.append(f_hat)
    in_specs.append(pl.BlockSpec((s_prev, _BB, _D), lambda b: (0, b, 0)))
    args.append(q)
    fout, r = pl.pallas_call(
        body,
        grid=(B // _BB,),
        in_specs=in_specs,
        out_specs=[
            pl.BlockSpec((_BB, 64, _D), lambda b: (b, 0, 0)),
            pl.BlockSpec((s, _BB, _D), lambda b: (0, b, 0)),
        ],
        out_shape=[
            jax.ShapeDtypeStruct((B, 64, _D), jnp.float32),
            jax.ShapeDtypeStruct((s, B, _D), jnp.float32),
        ],
    )(*args)
    return fout, r


def _final_body(z_ref, f_ref, q_ref, fout_ref, ls_ref):
    acc = jnp.zeros((_BB, _D), jnp.float32)
    for t in range(64):
        fo = f_ref[:, t, :] + q_ref[t, :, :]
        fout_ref[:, t, :] = fo
        dd = fo - z_ref[:, t, :]
        acc = acc + dd * dd
    ls_ref[0, 0, :] = jnp.sum(acc, axis=0)


def _final(z, f_hat, q):
    B = z.shape[0]
    fout, ls = pl.pallas_call(
        _final_body,
        grid=(B // _BB,),
        in_specs=[
            pl.BlockSpec((_BB, 64, _D), lambda b: (b, 0, 0)),
            pl.BlockSpec((_BB, 64, _D), lambda b: (b, 0, 0)),
            pl.BlockSpec((64, _BB, _D), lambda b: (0, b, 0)),
        ],
        out_specs=[
            pl.BlockSpec((_BB, 64, _D), lambda b: (b, 0, 0)),
            pl.BlockSpec((1, 1, _D), lambda b: (b, 0, 0)),
        ],
        out_shape=[
            jax.ShapeDtypeStruct((B, 64, _D), jnp.float32),
            jax.ShapeDtypeStruct((B // _BB, 1, _D), jnp.float32),
        ],
    )(z, f_hat, q)
    loss = jnp.sum(ls) / np.float32(B * 64 * _D)
    return fout, loss


def _sc_gather(codebook, idx, N):
    info = plsc.get_sparse_core_info()
    NW = info.num_cores * info.num_subcores
    b_per_w = N // NW
    chunk = min(128, b_per_w)
    nch = b_per_w // chunk
    mesh = plsc.VectorSubcoreMesh(core_axis_name="c", subcore_axis_name="s")

    @functools.partial(
        pl.kernel,
        out_type=jax.ShapeDtypeStruct((N, _D), jnp.float32),
        mesh=mesh,
        scratch_types=[
            pltpu.VMEM((chunk,), jnp.int32),
            pltpu.VMEM((chunk, _D), jnp.float32),
            pltpu.SemaphoreType.DMA,
        ],
    )
    def gth(cb_hbm, idx_hbm, out_hbm, idx_v, rows_v, sem):
        wid = lax.axis_index("s") * info.num_cores + lax.axis_index("c")
        base = wid * b_per_w
        for c in range(nch):
            off = base + c * chunk
            pltpu.sync_copy(idx_hbm.at[pl.ds(off, chunk)], idx_v)
            pltpu.async_copy(cb_hbm.at[idx_v], rows_v, sem).wait()
            pltpu.sync_copy(rows_v, out_hbm.at[pl.ds(off, chunk)])

    return gth(codebook, idx)


def kernel(z, codebook):
    B, S_last, D = z.shape
    c2 = jnp.sum(codebook * codebook, axis=-1).reshape(1, _K)
    f_hat = None
    q = None
    for k, s in enumerate(_SCALES):
        if k == 0:
            r3 = _down_first(z, s)
        else:
            f_hat, r3 = _glue(z, f_hat, q, _SCALES[k - 1], s)
        idx = _nn_idx(r3.reshape(s * B, _D), codebook, c2)
        q = _sc_gather(codebook, idx, s * B).reshape(s, B, _D)
    f_hat, loss = _final(z, f_hat, q)
    return f_hat, loss, loss


# speedup vs baseline: 1.5386x; 1.0269x over previous
import functools

import numpy as np
import jax
import jax.numpy as jnp
from jax import lax
from jax.experimental import pallas as pl
from jax.experimental.pallas import tpu as pltpu
from jax.experimental.pallas import tpu_sc as plsc

_SCALES = [1, 2, 4, 8, 16, 32, 64]
_K = 8192
_D = 256
_BM = 256
_BB = 8


def _interp_wts(S_in, size):
    pos = (np.arange(size, dtype=np.float32) + np.float32(0.5)) \
        * np.float32(S_in / size) - np.float32(0.5)
    pos = np.clip(pos, np.float32(0.0), np.float32(S_in - 1))
    lo = np.floor(pos).astype(np.int32)
    hi = np.minimum(lo + 1, S_in - 1).astype(np.int32)
    w = (pos - lo.astype(np.float32)).astype(np.float32)
    return lo, hi, w


def _argmin_rows(d):
    m = jnp.min(d, axis=1, keepdims=True)
    ii = lax.broadcasted_iota(jnp.int32, d.shape, 1)
    return jnp.min(jnp.where(d == m, ii, _K), axis=1, keepdims=True)


def _nn_body(r_ref, cb_ref, c2_ref, out_ref):
    r = r_ref[...]
    r2 = jnp.sum(r * r, axis=1, keepdims=True)
    dot = lax.dot_general(r, cb_ref[...], (((1,), (1,)), ((), ())),
                          preferred_element_type=jnp.float32)
    d = (r2 - 2.0 * dot) + c2_ref[...]
    out_ref[...] = _argmin_rows(d)


def _nn_idx(r_flat, codebook, c2):
    N = r_flat.shape[0]
    out = pl.pallas_call(
        _nn_body,
        grid=(N // _BM,),
        in_specs=[
            pl.BlockSpec((_BM, _D), lambda i: (i, 0)),
            pl.BlockSpec((_K, _D), lambda i: (0, 0)),
            pl.BlockSpec((1, _K), lambda i: (0, 0)),
        ],
        out_specs=pl.BlockSpec((_BM, 1), lambda i: (i, 0)),
        out_shape=jax.ShapeDtypeStruct((N, 1), jnp.int32),
        compiler_params=pltpu.CompilerParams(
            dimension_semantics=("arbitrary",)),
    )(r_flat, codebook, c2)
    return out.reshape(N)


def _down0_body(s, down, z_ref, r_ref):
    lo, hi, w = down
    for i in range(s):
        wi = np.float32(w[i])
        wi1 = np.float32(1.0) - wi
        r_ref[i, :, :] = z_ref[:, int(lo[i]), :] * wi1 \
            + z_ref[:, int(hi[i]), :] * wi


def _down_first(z, s):
    B = z.shape[0]
    body = functools.partial(_down0_body, s, _interp_wts(64, s))
    return pl.pallas_call(
        body,
        grid=(B // _BB,),
        in_specs=[pl.BlockSpec((_BB, 64, _D), lambda b: (b, 0, 0))],
        out_specs=pl.BlockSpec((s, _BB, _D), lambda b: (0, b, 0)),
        out_shape=jax.ShapeDtypeStruct((s, B, _D), jnp.float32),
    )(z)


def _glue_body(s_prev, s, up, down, has_f, z_ref, *refs):
    if has_f:
        f_ref, q_ref, fout_ref, r_ref = refs
    else:
        q_ref, fout_ref, r_ref = refs
        f_ref = None
    lo_u, hi_u, w_u = up
    lo_d, hi_d, w_d = down
    for t in range(64):
        wt = np.float32(w_u[t])
        wt1 = np.float32(1.0) - wt
        y = q_ref[int(lo_u[t]), :, :] * wt1 + q_ref[int(hi_u[t]), :, :] * wt
        fout_ref[:, t, :] = y if f_ref is None else f_ref[:, t, :] + y
    for i in range(s):
        wi = np.float32(w_d[i])
        wi1 = np.float32(1.0) - wi
        ra = z_ref[:, int(lo_d[i]), :] - fout_ref[:, int(lo_d[i]), :]
        rb = z_ref[:, int(hi_d[i]), :] - fout_ref[:, int(hi_d[i]), :]
        r_ref[i, :, :] = ra * wi1 + rb * wi


def _glue(z, f_hat, q, s_prev, s):
    B = z.shape[0]
    up = _interp_wts(s_prev, 64)
    down = _interp_wts(64, s)
    has_f = f_hat is not None
    body = functools.partial(_glue_body, s_prev, s, up, down, has_f)
    in_specs = [pl.BlockSpec((_BB, 64, _D), lambda b: (b, 0, 0))]
    args = [z]
    if has_f:
        in_specs.append(pl.BlockSpec((_BB, 64, _D), lambda b: (b, 0, 0)))
        args.append(f_hat)
    in_specs.append(pl.BlockSpec((s_prev, _BB, _D), lambda b: (0, b, 0)))
    args.append(q)
    fout, r = pl.pallas_call(
        body,
        grid=(B // _BB,),
        in_specs=in_specs,
        out_specs=[
            pl.BlockSpec((_BB, 64, _D), lambda b: (b, 0, 0)),
            pl.BlockSpec((s, _BB, _D), lambda b: (0, b, 0)),
        ],
        out_shape=[
            jax.ShapeDtypeStruct((B, 64, _D), jnp.float32),
            jax.ShapeDtypeStruct((s, B, _D), jnp.float32),
        ],
    )(*args)
    return fout, r


def _merged_body(s_prev, s, bbs, up, down, grouped,
                 z_ref, f_ref, q_ref, cb_ref, c2_ref,
                 fout_ref, idx_ref, r_s):
    lo_u, hi_u, w_u = up
    lo_d, hi_d, w_d = down

    def qrow(j):
        return q_ref[0, j, 0, :, :] if grouped else q_ref[j, :, :]

    for t in range(64):
        wt = np.float32(w_u[t])
        wt1 = np.float32(1.0) - wt
        y = qrow(int(lo_u[t])) * wt1 + qrow(int(hi_u[t])) * wt
        fout_ref[:, t, :] = f_ref[:, t, :] + y
    for i in range(s):
        if s == 64:
            rr = z_ref[:, i, :] - fout_ref[:, i, :]
        else:
            wi = np.float32(w_d[i])
            wi1 = np.float32(1.0) - wi
            ra = z_ref[:, int(lo_d[i]), :] - fout_ref[:, int(lo_d[i]), :]
            rb = z_ref[:, int(hi_d[i]), :] - fout_ref[:, int(hi_d[i]), :]
            rr = ra * wi1 + rb * wi
        r_s[i * bbs:(i + 1) * bbs, :] = rr
    r = r_s[...]
    r2 = jnp.sum(r * r, axis=1, keepdims=True)
    dot = lax.dot_general(r, cb_ref[...], (((1,), (1,)), ((), ())),
                          preferred_element_type=jnp.float32)
    d = (r2 - 2.0 * dot) + c2_ref[...]
    idx_ref[...] = _argmin_rows(d)


def _merged(z, f_hat, q, codebook, c2, s_prev, s, grouped):
    B = z.shape[0]
    bbs = 256 // s
    G = B // bbs
    up = _interp_wts(s_prev, 64)
    down = _interp_wts(64, s)
    body = functools.partial(_merged_body, s_prev, s, bbs, up, down, grouped)
    if grouped:
        q_spec = pl.BlockSpec((1, s_prev, 1, bbs, _D),
                              lambda b: (b // 2, 0, b % 2, 0, 0))
    else:
        q_spec = pl.BlockSpec((s_prev, bbs, _D), lambda b: (0, b, 0))
    fout, idx = pl.pallas_call(
        body,
        grid=(G,),
        in_specs=[
            pl.BlockSpec((bbs, 64, _D), lambda b: (b, 0, 0)),
            pl.BlockSpec((bbs, 64, _D), lambda b: (b, 0, 0)),
            q_spec,
            pl.BlockSpec((_K, _D), lambda b: (0, 0)),
            pl.BlockSpec((1, _K), lambda b: (0, 0)),
        ],
        out_specs=[
            pl.BlockSpec((bbs, 64, _D), lambda b: (b, 0, 0)),
            pl.BlockSpec((_BM, 1), lambda b: (b, 0)),
        ],
        out_shape=[
            jax.ShapeDtypeStruct((B, 64, _D), jnp.float32),
            jax.ShapeDtypeStruct((s * B, 1), jnp.int32),
        ],
        scratch_shapes=[pltpu.VMEM((_BM, _D), jnp.float32)],
        compiler_params=pltpu.CompilerParams(
            dimension_semantics=("arbitrary",)),
    )(z, f_hat, q, codebook, c2)
    return fout, idx.reshape(s * B)


def _final_body(z_ref, f_ref, q_ref, fout_ref, ls_ref):
    acc = jnp.zeros((4, _D), jnp.float32)
    for t in range(64):
        qt = jnp.concatenate([q_ref[0, t, 0, :, :], q_ref[0, t, 1, :, :]],
                             axis=0)
        fo = f_ref[:, t, :] + qt
        fout_ref[:, t, :] = fo
        dd = fo - z_ref[:, t, :]
        acc = acc + dd * dd
    ls_ref[0, 0, :] = jnp.sum(acc, axis=0)


def _final(z, f_hat, q):
    B = z.shape[0]
    fout, ls = pl.pallas_call(
        _final_body,
        grid=(B // 4,),
        in_specs=[
            pl.BlockSpec((4, 64, _D), lambda b: (b, 0, 0)),
            pl.BlockSpec((4, 64, _D), lambda b: (b, 0, 0)),
            pl.BlockSpec((1, 64, 2, 2, _D), lambda b: (b, 0, 0, 0, 0)),
        ],
        out_specs=[
            pl.BlockSpec((4, 64, _D), lambda b: (b, 0, 0)),
            pl.BlockSpec((1, 1, _D), lambda b: (b, 0, 0)),
        ],
        out_shape=[
            jax.ShapeDtypeStruct((B, 64, _D), jnp.float32),
            jax.ShapeDtypeStruct((B // 4, 1, _D), jnp.float32),
        ],
    )(z, f_hat, q)
    loss = jnp.sum(ls) / np.float32(B * 64 * _D)
    return fout, loss


def _sc_gather(codebook, idx, N):
    info = plsc.get_sparse_core_info()
    NW = info.num_cores * info.num_subcores
    b_per_w = N // NW
    chunk = min(128, b_per_w)
    nch = b_per_w // chunk
    mesh = plsc.VectorSubcoreMesh(core_axis_name="c", subcore_axis_name="s")

    @functools.partial(
        pl.kernel,
        out_type=jax.ShapeDtypeStruct((N, _D), jnp.float32),
        mesh=mesh,
        scratch_types=[
            pltpu.VMEM((2, chunk), jnp.int32),
            pltpu.VMEM((2, chunk, _D), jnp.float32),
            pltpu.SemaphoreType.DMA,
            pltpu.SemaphoreType.DMA,
            pltpu.SemaphoreType.DMA,
            pltpu.SemaphoreType.DMA,
        ],
    )
    def gth(cb_hbm, idx_hbm, out_hbm, idx_v, rows_v, g0, g1, o0, o1):
        wid = lax.axis_index("s") * info.num_cores + lax.axis_index("c")
        base = wid * b_per_w
        gsem = [g0, g1]
        osem = [o0, o1]
        hg = [None] * nch
        ho = [None] * nch
        for c in range(nch):
            buf = c & 1
            if c >= 2:
                ho[c - 2].wait()
            pltpu.sync_copy(idx_hbm.at[pl.ds(base + c * chunk, chunk)],
                            idx_v.at[buf])
            hg[c] = pltpu.async_copy(cb_hbm.at[idx_v.at[buf]],
                                     rows_v.at[buf], gsem[buf])
            if c >= 1:
                pbuf = (c - 1) & 1
                hg[c - 1].wait()
                ho[c - 1] = pltpu.async_copy(
                    rows_v.at[pbuf],
                    out_hbm.at[pl.ds(base + (c - 1) * chunk, chunk)],
                    osem[pbuf])
        lbuf = (nch - 1) & 1
        hg[nch - 1].wait()
        ho[nch - 1] = pltpu.async_copy(
            rows_v.at[lbuf],
            out_hbm.at[pl.ds(base + (nch - 1) * chunk, chunk)],
            osem[lbuf])
        if nch >= 2:
            ho[nch - 2].wait()
        ho[nch - 1].wait()

    return gth(codebook, idx)


def kernel(z, codebook):
    B, S_last, D = z.shape
    c2 = jnp.sum(codebook * codebook, axis=-1).reshape(1, _K)
    f_hat = None
    q = None
    for k, s in enumerate(_SCALES):
        s_prev = _SCALES[k - 1] if k else None
        if s < 8:
            if k == 0:
                r3 = _down_first(z, s)
            else:
                f_hat, r3 = _glue(z, f_hat, q, s_prev, s)
            idx = _nn_idx(r3.reshape(s * B, _D), codebook, c2)
            q = _sc_gather(codebook, idx, s * B).reshape(s, B, _D)
        else:
            grouped = s > 8
            f_hat, idx = _merged(z, f_hat, q, codebook, c2, s_prev, s,
                                 grouped)
            bbs = 256 // s
            q = _sc_gather(codebook, idx, s * B) \
                .reshape(B // bbs, s, 2, bbs // 2, _D)
    f_hat, loss = _final(z, f_hat, q)
    return f_hat, loss, loss


# speedup vs baseline: 1.5425x; 1.0025x over previous
import functools

import numpy as np
import jax
import jax.numpy as jnp
from jax import lax
from jax.experimental import pallas as pl
from jax.experimental.pallas import tpu as pltpu
from jax.experimental.pallas import tpu_sc as plsc

_SCALES = [1, 2, 4, 8, 16, 32, 64]
_K = 8192
_D = 256
_BM = 256
_BB = 8
_HI = jax.lax.Precision.HIGHEST


def _interp_wts(S_in, size):
    pos = (np.arange(size, dtype=np.float32) + np.float32(0.5)) \
        * np.float32(S_in / size) - np.float32(0.5)
    pos = np.clip(pos, np.float32(0.0), np.float32(S_in - 1))
    lo = np.floor(pos).astype(np.int32)
    hi = np.minimum(lo + 1, S_in - 1).astype(np.int32)
    w = (pos - lo.astype(np.float32)).astype(np.float32)
    return lo, hi, w


def _argmin_rows(d):
    m = jnp.min(d, axis=1, keepdims=True)
    ii = lax.broadcasted_iota(jnp.int32, d.shape, 1)
    return jnp.min(jnp.where(d == m, ii, _K), axis=1, keepdims=True)


def _nn_body(r_ref, cb_ref, c2_ref, out_ref):
    r = r_ref[...]
    r2 = jnp.sum(r * r, axis=1, keepdims=True)
    dot = lax.dot_general(r, cb_ref[...], (((1,), (1,)), ((), ())),
                          preferred_element_type=jnp.float32)
    d = (r2 - 2.0 * dot) + c2_ref[...]
    out_ref[...] = _argmin_rows(d)


def _nn_idx(r_flat, codebook, c2):
    N = r_flat.shape[0]
    out = pl.pallas_call(
        _nn_body,
        grid=(N // _BM,),
        in_specs=[
            pl.BlockSpec((_BM, _D), lambda i: (i, 0)),
            pl.BlockSpec((_K, _D), lambda i: (0, 0)),
            pl.BlockSpec((1, _K), lambda i: (0, 0)),
        ],
        out_specs=pl.BlockSpec((_BM, 1), lambda i: (i, 0)),
        out_shape=jax.ShapeDtypeStruct((N, 1), jnp.int32),
        compiler_params=pltpu.CompilerParams(
            dimension_semantics=("arbitrary",)),
    )(r_flat, codebook, c2)
    return out.reshape(N)


def _down0_body(s, down, z_ref, r_ref):
    lo, hi, w = down
    for i in range(s):
        wi = np.float32(w[i])
        wi1 = np.float32(1.0) - wi
        r_ref[i, :, :] = z_ref[int(lo[i]), :, :] * wi1 \
            + z_ref[int(hi[i]), :, :] * wi


def _down_first(zt, s):
    B = zt.shape[1]
    body = functools.partial(_down0_body, s, _interp_wts(64, s))
    return pl.pallas_call(
        body,
        grid=(B // _BB,),
        in_specs=[pl.BlockSpec((64, _BB, _D), lambda b: (0, b, 0))],
        out_specs=pl.BlockSpec((s, _BB, _D), lambda b: (0, b, 0)),
        out_shape=jax.ShapeDtypeStruct((s, B, _D), jnp.float32),
    )(zt)


def _glue_body(s_prev, s, up, down, has_f, z_ref, *refs):
    if has_f:
        f_ref, q_ref, fout_ref, r_ref = refs
    else:
        q_ref, fout_ref, r_ref = refs
        f_ref = None
    lo_u, hi_u, w_u = up
    lo_d, hi_d, w_d = down
    for t in range(64):
        wt = np.float32(w_u[t])
        wt1 = np.float32(1.0) - wt
        y = q_ref[int(lo_u[t]), :, :] * wt1 + q_ref[int(hi_u[t]), :, :] * wt
        fout_ref[t, :, :] = y if f_ref is None else f_ref[t, :, :] + y
    for i in range(s):
        wi = np.float32(w_d[i])
        wi1 = np.float32(1.0) - wi
        ra = z_ref[int(lo_d[i]), :, :] - fout_ref[int(lo_d[i]), :, :]
        rb = z_ref[int(hi_d[i]), :, :] - fout_ref[int(hi_d[i]), :, :]
        r_ref[i, :, :] = ra * wi1 + rb * wi


def _glue(zt, f_hat, q, s_prev, s):
    B = zt.shape[1]
    up = _interp_wts(s_prev, 64)
    down = _interp_wts(64, s)
    has_f = f_hat is not None
    body = functools.partial(_glue_body, s_prev, s, up, down, has_f)
    in_specs = [pl.BlockSpec((64, _BB, _D), lambda b: (0, b, 0))]
    args = [zt]
    if has_f:
        in_specs.append(pl.BlockSpec((64, _BB, _D), lambda b: (0, b, 0)))
        args.append(f_hat)
    in_specs.append(pl.BlockSpec((s_prev, _BB, _D), lambda b: (0, b, 0)))
    args.append(q)
    fout, r = pl.pallas_call(
        body,
        grid=(B // _BB,),
        in_specs=in_specs,
        out_specs=[
            pl.BlockSpec((64, _BB, _D), lambda b: (0, b, 0)),
            pl.BlockSpec((s, _BB, _D), lambda b: (0, b, 0)),
        ],
        out_shape=[
            jax.ShapeDtypeStruct((64, B, _D), jnp.float32),
            jax.ShapeDtypeStruct((s, B, _D), jnp.float32),
        ],
    )(*args)
    return fout, r


def _merged_body(s_prev, s, bbs, up, down, grouped, consts,
                 z_ref, f_ref, q_ref, cb_ref, c2_ref, *rest):
    lo_u, hi_u, w_u = up
    lo_d, hi_d, w_d = down
    if consts:
        plo_ref, phi_ref, wlo_ref, whi_ref = rest[:4]
        rest = rest[4:]
    fout_ref, idx_ref = rest[:2]
    r_s = rest[2] if len(rest) > 2 else None

    def qrow(j):
        return q_ref[0, j, 0, :, :] if grouped else q_ref[j, :, :]

    if s == 64:
        qv = q_ref[...].reshape(s_prev * bbs, _D)
        qlo = lax.dot_general(plo_ref[...], qv, (((1,), (0,)), ((), ())),
                              preferred_element_type=jnp.float32,
                              precision=_HI)
        qhi = lax.dot_general(phi_ref[...], qv, (((1,), (0,)), ((), ())),
                              preferred_element_type=jnp.float32,
                              precision=_HI)
        y = qlo * wlo_ref[...] + qhi * whi_ref[...]
        fout = f_ref[...].reshape(64 * bbs, _D) + y
        fout_ref[...] = fout.reshape(fout_ref.shape)
        r = z_ref[...].reshape(64 * bbs, _D) - fout
    else:
        for t in range(64):
            wt = np.float32(w_u[t])
            wt1 = np.float32(1.0) - wt
            y = qrow(int(lo_u[t])) * wt1 + qrow(int(hi_u[t])) * wt
            fout_ref[t, :, :] = f_ref[t, :, :] + y
        for i in range(s):
            wi = np.float32(w_d[i])
            wi1 = np.float32(1.0) - wi
            ra = z_ref[int(lo_d[i]), :, :] - fout_ref[int(lo_d[i]), :, :]
            rb = z_ref[int(hi_d[i]), :, :] - fout_ref[int(hi_d[i]), :, :]
            r_s[i * bbs:(i + 1) * bbs, :] = ra * wi1 + rb * wi
        r = r_s[...]
    r2 = jnp.sum(r * r, axis=1, keepdims=True)
    dot = lax.dot_general(r, cb_ref[...], (((1,), (1,)), ((), ())),
                          preferred_element_type=jnp.float32)
    d = (r2 - 2.0 * dot) + c2_ref[...]
    idx_ref[...] = _argmin_rows(d)


def _merged(zt, f_hat, q, codebook, c2, s_prev, s, grouped):
    B = zt.shape[1]
    bbs = 256 // s
    G = B // bbs
    up = _interp_wts(s_prev, 64)
    down = _interp_wts(64, s)
    consts = s == 64
    body = functools.partial(_merged_body, s_prev, s, bbs, up, down,
                             grouped, consts)
    if grouped:
        q_spec = pl.BlockSpec((1, s_prev, 1, bbs, _D),
                              lambda b: (b // 2, 0, b % 2, 0, 0))
    else:
        q_spec = pl.BlockSpec((s_prev, bbs, _D), lambda b: (0, b, 0))
    if bbs < 8:
        zf_spec = pl.BlockSpec((64, 1, bbs, _D), lambda b: (0, b, 0, 0))
        zt = zt.reshape(64, B // bbs, bbs, _D)
        f_hat = f_hat.reshape(64, B // bbs, bbs, _D)
        fout_shape = (64, B // bbs, bbs, _D)
    else:
        zf_spec = pl.BlockSpec((64, bbs, _D), lambda b: (0, b, 0))
        fout_shape = (64, B, _D)
    in_specs = [
        zf_spec,
        zf_spec,
        q_spec,
        pl.BlockSpec((_K, _D), lambda b: (0, 0)),
        pl.BlockSpec((1, _K), lambda b: (0, 0)),
    ]
    args = [zt, f_hat, q, codebook, c2]
    if consts:
        lo_u, hi_u, w_u = up
        nrow = 64 * bbs
        ncol = s_prev * bbs
        plo = np.zeros((nrow, ncol), np.float32)
        phi = np.zeros((nrow, ncol), np.float32)
        wlo = np.empty((nrow, 1), np.float32)
        whi = np.empty((nrow, 1), np.float32)
        for t in range(64):
            for u in range(bbs):
                row = t * bbs + u
                plo[row, int(lo_u[t]) * bbs + u] = 1.0
                phi[row, int(hi_u[t]) * bbs + u] = 1.0
                whi[row, 0] = w_u[t]
                wlo[row, 0] = np.float32(1.0) - w_u[t]
        in_specs += [
            pl.BlockSpec((nrow, ncol), lambda b: (0, 0)),
            pl.BlockSpec((nrow, ncol), lambda b: (0, 0)),
            pl.BlockSpec((nrow, 1), lambda b: (0, 0)),
            pl.BlockSpec((nrow, 1), lambda b: (0, 0)),
        ]
        args += [jnp.asarray(plo), jnp.asarray(phi),
                 jnp.asarray(wlo), jnp.asarray(whi)]
    fout, idx = pl.pallas_call(
        body,
        grid=(G,),
        in_specs=in_specs,
        out_specs=[
            zf_spec,
            pl.BlockSpec((_BM, 1), lambda b: (b, 0)),
        ],
        out_shape=[
            jax.ShapeDtypeStruct(fout_shape, jnp.float32),
            jax.ShapeDtypeStruct((s * B, 1), jnp.int32),
        ],
        scratch_shapes=([] if consts
                        else [pltpu.VMEM((_BM, _D), jnp.float32)]),
        compiler_params=pltpu.CompilerParams(
            dimension_semantics=("arbitrary",)),
    )(*args)
    return fout, idx.reshape(s * B)


def _final_body(z_ref, f_ref, q_ref, fout_ref, ls_ref):
    fo = f_ref[...].reshape(256, _D) + q_ref[...].reshape(256, _D)
    fout_ref[...] = fo.reshape(fout_ref.shape)
    dd = fo - z_ref[...].reshape(256, _D)
    ls_ref[0, 0, :] = jnp.sum(dd * dd, axis=0)


def _final(zt, f_hat, q):
    B = zt.shape[1] * zt.shape[2]
    zf_spec = pl.BlockSpec((64, 1, 4, _D), lambda b: (0, b, 0, 0))
    fout, ls = pl.pallas_call(
        _final_body,
        grid=(B // 4,),
        in_specs=[
            zf_spec,
            zf_spec,
            pl.BlockSpec((1, 64, 2, 2, _D), lambda b: (b, 0, 0, 0, 0)),
        ],
        out_specs=[
            zf_spec,
            pl.BlockSpec((1, 1, _D), lambda b: (b, 0, 0)),
        ],
        out_shape=[
            jax.ShapeDtypeStruct((64, B // 4, 4, _D), jnp.float32),
            jax.ShapeDtypeStruct((B // 4, 1, _D), jnp.float32),
        ],
    )(zt, f_hat, q)
    loss = jnp.sum(ls) / np.float32(B * 64 * _D)
    return fout, loss


def _sc_gather(codebook, idx, N):
    info = plsc.get_sparse_core_info()
    NW = info.num_cores * info.num_subcores
    b_per_w = N // NW
    chunk = min(128, b_per_w)
    nch = b_per_w // chunk
    mesh = plsc.VectorSubcoreMesh(core_axis_name="c", subcore_axis_name="s")

    @functools.partial(
        pl.kernel,
        out_type=jax.ShapeDtypeStruct((N, _D), jnp.float32),
        mesh=mesh,
        scratch_types=[
            pltpu.VMEM((2, chunk), jnp.int32),
            pltpu.VMEM((2, chunk, _D), jnp.float32),
            pltpu.SemaphoreType.DMA,
            pltpu.SemaphoreType.DMA,
            pltpu.SemaphoreType.DMA,
            pltpu.SemaphoreType.DMA,
        ],
    )
    def gth(cb_hbm, idx_hbm, out_hbm, idx_v, rows_v, g0, g1, o0, o1):
        wid = lax.axis_index("s") * info.num_cores + lax.axis_index("c")
        base = wid * b_per_w
        gsem = [g0, g1]
        osem = [o0, o1]
        hg = [None] * nch
        ho = [None] * nch
        for c in range(nch):
            buf = c & 1
            if c >= 2:
                ho[c - 2].wait()
            pltpu.sync_copy(idx_hbm.at[pl.ds(base + c * chunk, chunk)],
                            idx_v.at[buf])
            hg[c] = pltpu.async_copy(cb_hbm.at[idx_v.at[buf]],
                                     rows_v.at[buf], gsem[buf])
            if c >= 1:
                pbuf = (c - 1) & 1
                hg[c - 1].wait()
                ho[c - 1] = pltpu.async_copy(
                    rows_v.at[pbuf],
                    out_hbm.at[pl.ds(base + (c - 1) * chunk, chunk)],
                    osem[pbuf])
        lbuf = (nch - 1) & 1
        hg[nch - 1].wait()
        ho[nch - 1] = pltpu.async_copy(
            rows_v.at[lbuf],
            out_hbm.at[pl.ds(base + (nch - 1) * chunk, chunk)],
            osem[lbuf])
        if nch >= 2:
            ho[nch - 2].wait()
        ho[nch - 1].wait()

    return gth(codebook, idx)


def kernel(z, codebook):
    B, S_last, D = z.shape
    c2 = jnp.sum(codebook * codebook, axis=-1).reshape(1, _K)
    zt = jnp.transpose(z, (1, 0, 2))
    f_hat = None
    q = None
    for k, s in enumerate(_SCALES):
        s_prev = _SCALES[k - 1] if k else None
        if s < 8:
            if k == 0:
                r3 = _down_first(zt, s)
            else:
                f_hat, r3 = _glue(zt, f_hat, q, s_prev, s)
            idx = _nn_idx(r3.reshape(s * B, _D), codebook, c2)
            q = _sc_gather(codebook, idx, s * B).reshape(s, B, _D)
        else:
            grouped = s > 8
            f_hat, idx = _merged(zt, f_hat, q, codebook, c2, s_prev, s,
                                 grouped)
            bbs = 256 // s
            q = _sc_gather(codebook, idx, s * B) \
                .reshape(B // bbs, s, 2, bbs // 2, _D)
    f_hat, loss = _final(zt.reshape(64, B // 4, 4, _D), f_hat, q)
    return jnp.transpose(f_hat.reshape(64, B, _D), (1, 0, 2)), loss, loss


# speedup vs baseline: 1.7724x; 1.1490x over previous
import functools

import numpy as np
import jax
import jax.numpy as jnp
from jax import lax
from jax.experimental import pallas as pl
from jax.experimental.pallas import tpu as pltpu
from jax.experimental.pallas import tpu_sc as plsc

_SCALES = [1, 2, 4, 8, 16, 32, 64]
_K = 8192
_D = 256
_BM = 256
_BB = 8
_HI = jax.lax.Precision.HIGHEST


def _interp_wts(S_in, size):
    pos = (np.arange(size, dtype=np.float32) + np.float32(0.5)) \
        * np.float32(S_in / size) - np.float32(0.5)
    pos = np.clip(pos, np.float32(0.0), np.float32(S_in - 1))
    lo = np.floor(pos).astype(np.int32)
    hi = np.minimum(lo + 1, S_in - 1).astype(np.int32)
    w = (pos - lo.astype(np.float32)).astype(np.float32)
    return lo, hi, w


def _argmin_rows(d):
    m = jnp.min(d, axis=1, keepdims=True)
    ii = lax.broadcasted_iota(jnp.int32, d.shape, 1)
    return jnp.min(jnp.where(d == m, ii, _K), axis=1, keepdims=True)


def _nn_body(r_ref, cb_ref, c2_ref, out_ref):
    r = r_ref[...]
    r2 = jnp.sum(r * r, axis=1, keepdims=True)
    dot = lax.dot_general(r, cb_ref[...], (((1,), (1,)), ((), ())),
                          preferred_element_type=jnp.float32)
    d = (r2 - 2.0 * dot) + c2_ref[...]
    out_ref[...] = _argmin_rows(d)


def _nn_idx(r_flat, codebook, c2):
    N = r_flat.shape[0]
    out = pl.pallas_call(
        _nn_body,
        grid=(N // _BM,),
        in_specs=[
            pl.BlockSpec((_BM, _D), lambda i: (i, 0)),
            pl.BlockSpec((_K, _D), lambda i: (0, 0)),
            pl.BlockSpec((1, _K), lambda i: (0, 0)),
        ],
        out_specs=pl.BlockSpec((_BM, 1), lambda i: (i, 0)),
        out_shape=jax.ShapeDtypeStruct((N, 1), jnp.int32),
        compiler_params=pltpu.CompilerParams(
            dimension_semantics=("arbitrary",)),
    )(r_flat, codebook, c2)
    return out.reshape(N)


def _down0_body(s, down, z_ref, r_ref):
    lo, hi, w = down
    for i in range(s):
        wi = np.float32(w[i])
        wi1 = np.float32(1.0) - wi
        r_ref[i, :, :] = z_ref[int(lo[i]), :, :] * wi1 \
            + z_ref[int(hi[i]), :, :] * wi


def _down_first(zt, s):
    B = zt.shape[1]
    body = functools.partial(_down0_body, s, _interp_wts(64, s))
    return pl.pallas_call(
        body,
        grid=(B // _BB,),
        in_specs=[pl.BlockSpec((64, _BB, _D), lambda b: (0, b, 0))],
        out_specs=pl.BlockSpec((s, _BB, _D), lambda b: (0, b, 0)),
        out_shape=jax.ShapeDtypeStruct((s, B, _D), jnp.float32),
    )(zt)


def _glue_body(s_prev, s, up, down, has_f, z_ref, *refs):
    if has_f:
        f_ref, q_ref, fout_ref, r_ref = refs
    else:
        q_ref, fout_ref, r_ref = refs
        f_ref = None
    lo_u, hi_u, w_u = up
    lo_d, hi_d, w_d = down
    for t in range(64):
        wt = np.float32(w_u[t])
        wt1 = np.float32(1.0) - wt
        y = q_ref[int(lo_u[t]), :, :] * wt1 + q_ref[int(hi_u[t]), :, :] * wt
        fout_ref[t, :, :] = y if f_ref is None else f_ref[t, :, :] + y
    for i in range(s):
        wi = np.float32(w_d[i])
        wi1 = np.float32(1.0) - wi
        ra = z_ref[int(lo_d[i]), :, :] - fout_ref[int(lo_d[i]), :, :]
        rb = z_ref[int(hi_d[i]), :, :] - fout_ref[int(hi_d[i]), :, :]
        r_ref[i, :, :] = ra * wi1 + rb * wi


def _glue(zt, f_hat, q, s_prev, s):
    B = zt.shape[1]
    up = _interp_wts(s_prev, 64)
    down = _interp_wts(64, s)
    has_f = f_hat is not None
    body = functools.partial(_glue_body, s_prev, s, up, down, has_f)
    in_specs = [pl.BlockSpec((64, _BB, _D), lambda b: (0, b, 0))]
    args = [zt]
    if has_f:
        in_specs.append(pl.BlockSpec((64, _BB, _D), lambda b: (0, b, 0)))
        args.append(f_hat)
    in_specs.append(pl.BlockSpec((s_prev, _BB, _D), lambda b: (0, b, 0)))
    args.append(q)
    fout, r = pl.pallas_call(
        body,
        grid=(B // _BB,),
        in_specs=in_specs,
        out_specs=[
            pl.BlockSpec((64, _BB, _D), lambda b: (0, b, 0)),
            pl.BlockSpec((s, _BB, _D), lambda b: (0, b, 0)),
        ],
        out_shape=[
            jax.ShapeDtypeStruct((64, B, _D), jnp.float32),
            jax.ShapeDtypeStruct((s, B, _D), jnp.float32),
        ],
    )(*args)
    return fout, r


def _merged_body(s_prev, s, bbs, up, down, grouped, consts,
                 z_ref, f_ref, q_ref, cb_ref, c2_ref, *rest):
    lo_u, hi_u, w_u = up
    lo_d, hi_d, w_d = down
    if consts:
        plo_ref, phi_ref, wlo_ref, whi_ref = rest[:4]
        rest = rest[4:]
    fout_ref, idx_ref = rest[:2]
    r_s = rest[2] if len(rest) > 2 else None

    def qrow(j):
        return q_ref[0, j, 0, :, :] if grouped else q_ref[j, :, :]

    if s == 64:
        qv = q_ref[...].reshape(s_prev * bbs, _D)
        qlo = lax.dot_general(plo_ref[...], qv, (((1,), (0,)), ((), ())),
                              preferred_element_type=jnp.float32,
                              precision=_HI)
        qhi = lax.dot_general(phi_ref[...], qv, (((1,), (0,)), ((), ())),
                              preferred_element_type=jnp.float32,
                              precision=_HI)
        y = qlo * wlo_ref[...] + qhi * whi_ref[...]
        fout = f_ref[...].reshape(64 * bbs, _D) + y
        fout_ref[...] = fout.reshape(fout_ref.shape)
        r = z_ref[...].reshape(64 * bbs, _D) - fout
    else:
        for t in range(64):
            wt = np.float32(w_u[t])
            wt1 = np.float32(1.0) - wt
            y = qrow(int(lo_u[t])) * wt1 + qrow(int(hi_u[t])) * wt
            fout_ref[t, :, :] = f_ref[t, :, :] + y
        for i in range(s):
            wi = np.float32(w_d[i])
            wi1 = np.float32(1.0) - wi
            ra = z_ref[int(lo_d[i]), :, :] - fout_ref[int(lo_d[i]), :, :]
            rb = z_ref[int(hi_d[i]), :, :] - fout_ref[int(hi_d[i]), :, :]
            r_s[i * bbs:(i + 1) * bbs, :] = ra * wi1 + rb * wi
        r = r_s[...]
    r2 = jnp.sum(r * r, axis=1, keepdims=True)
    dot = lax.dot_general(r, cb_ref[...], (((1,), (1,)), ((), ())),
                          preferred_element_type=jnp.float32)
    d = (r2 - 2.0 * dot) + c2_ref[...]
    idx_ref[...] = _argmin_rows(d)


def _merged(zt, f_hat, q, codebook, c2, s_prev, s, grouped):
    B = zt.shape[1]
    bbs = 512 // s
    G = B // bbs
    up = _interp_wts(s_prev, 64)
    down = _interp_wts(64, s)
    consts = s == 64
    body = functools.partial(_merged_body, s_prev, s, bbs, up, down,
                             grouped, consts)
    if grouped:
        q_spec = pl.BlockSpec((1, s_prev, 1, bbs, _D),
                              lambda b: (b // 2, 0, b % 2, 0, 0))
    else:
        q_spec = pl.BlockSpec((s_prev, bbs, _D), lambda b: (0, b, 0))
    if bbs < 8:
        zf_spec = pl.BlockSpec((64, 1, bbs, _D), lambda b: (0, b, 0, 0))
        zt = zt.reshape(64, B // bbs, bbs, _D)
        f_hat = f_hat.reshape(64, B // bbs, bbs, _D)
        fout_shape = (64, B // bbs, bbs, _D)
    else:
        zf_spec = pl.BlockSpec((64, bbs, _D), lambda b: (0, b, 0))
        fout_shape = (64, B, _D)
    in_specs = [
        zf_spec,
        zf_spec,
        q_spec,
        pl.BlockSpec((_K, _D), lambda b: (0, 0)),
        pl.BlockSpec((1, _K), lambda b: (0, 0)),
    ]
    args = [zt, f_hat, q, codebook, c2]
    if consts:
        lo_u, hi_u, w_u = up
        nrow = 64 * bbs
        ncol = s_prev * bbs
        plo = np.zeros((nrow, ncol), np.float32)
        phi = np.zeros((nrow, ncol), np.float32)
        wlo = np.empty((nrow, 1), np.float32)
        whi = np.empty((nrow, 1), np.float32)
        for t in range(64):
            for u in range(bbs):
                row = t * bbs + u
                plo[row, int(lo_u[t]) * bbs + u] = 1.0
                phi[row, int(hi_u[t]) * bbs + u] = 1.0
                whi[row, 0] = w_u[t]
                wlo[row, 0] = np.float32(1.0) - w_u[t]
        in_specs += [
            pl.BlockSpec((nrow, ncol), lambda b: (0, 0)),
            pl.BlockSpec((nrow, ncol), lambda b: (0, 0)),
            pl.BlockSpec((nrow, 1), lambda b: (0, 0)),
            pl.BlockSpec((nrow, 1), lambda b: (0, 0)),
        ]
        args += [jnp.asarray(plo), jnp.asarray(phi),
                 jnp.asarray(wlo), jnp.asarray(whi)]
    fout, idx = pl.pallas_call(
        body,
        grid=(G,),
        in_specs=in_specs,
        out_specs=[
            zf_spec,
            pl.BlockSpec((s * bbs, 1), lambda b: (b, 0)),
        ],
        out_shape=[
            jax.ShapeDtypeStruct(fout_shape, jnp.float32),
            jax.ShapeDtypeStruct((s * B, 1), jnp.int32),
        ],
        scratch_shapes=([] if consts
                        else [pltpu.VMEM((s * bbs, _D), jnp.float32)]),
        compiler_params=pltpu.CompilerParams(
            dimension_semantics=("arbitrary",)),
    )(*args)
    return fout, idx.reshape(s * B)


def _final_body(z_ref, f_ref, q_ref, fout_ref, ls_ref):
    fo = f_ref[...].reshape(512, _D) + q_ref[...].reshape(512, _D)
    fout_ref[...] = fo.reshape(fout_ref.shape)
    dd = fo - z_ref[...].reshape(512, _D)
    ls_ref[0, 0, :] = jnp.sum(dd * dd, axis=0)


def _final(zt, f_hat, q):
    B = zt.shape[1]
    zf_spec = pl.BlockSpec((64, 8, _D), lambda b: (0, b, 0))
    fout, ls = pl.pallas_call(
        _final_body,
        grid=(B // 8,),
        in_specs=[
            zf_spec,
            zf_spec,
            pl.BlockSpec((1, 64, 2, 4, _D), lambda b: (b, 0, 0, 0, 0)),
        ],
        out_specs=[
            zf_spec,
            pl.BlockSpec((1, 1, _D), lambda b: (b, 0, 0)),
        ],
        out_shape=[
            jax.ShapeDtypeStruct((64, B, _D), jnp.float32),
            jax.ShapeDtypeStruct((B // 8, 1, _D), jnp.float32),
        ],
    )(zt, f_hat, q)
    loss = jnp.sum(ls) / np.float32(B * 64 * _D)
    return fout, loss


def _sc_gather(codebook, idx, N):
    info = plsc.get_sparse_core_info()
    NW = info.num_cores * info.num_subcores
    b_per_w = N // NW
    chunk = min(128, b_per_w)
    nch = b_per_w // chunk
    mesh = plsc.VectorSubcoreMesh(core_axis_name="c", subcore_axis_name="s")

    @functools.partial(
        pl.kernel,
        out_type=jax.ShapeDtypeStruct((N, _D), jnp.float32),
        mesh=mesh,
        scratch_types=[
            pltpu.VMEM((2, chunk), jnp.int32),
            pltpu.VMEM((2, chunk, _D), jnp.float32),
            pltpu.SemaphoreType.DMA,
            pltpu.SemaphoreType.DMA,
            pltpu.SemaphoreType.DMA,
            pltpu.SemaphoreType.DMA,
        ],
    )
    def gth(cb_hbm, idx_hbm, out_hbm, idx_v, rows_v, g0, g1, o0, o1):
        wid = lax.axis_index("s") * info.num_cores + lax.axis_index("c")
        base = wid * b_per_w
        gsem = [g0, g1]
        osem = [o0, o1]
        hg = [None] * nch
        ho = [None] * nch
        for c in range(nch):
            buf = c & 1
            if c >= 2:
                ho[c - 2].wait()
            pltpu.sync_copy(idx_hbm.at[pl.ds(base + c * chunk, chunk)],
                            idx_v.at[buf])
            hg[c] = pltpu.async_copy(cb_hbm.at[idx_v.at[buf]],
                                     rows_v.at[buf], gsem[buf])
            if c >= 1:
                pbuf = (c - 1) & 1
                hg[c - 1].wait()
                ho[c - 1] = pltpu.async_copy(
                    rows_v.at[pbuf],
                    out_hbm.at[pl.ds(base + (c - 1) * chunk, chunk)],
                    osem[pbuf])
        lbuf = (nch - 1) & 1
        hg[nch - 1].wait()
        ho[nch - 1] = pltpu.async_copy(
            rows_v.at[lbuf],
            out_hbm.at[pl.ds(base + (nch - 1) * chunk, chunk)],
            osem[lbuf])
        if nch >= 2:
            ho[nch - 2].wait()
        ho[nch - 1].wait()

    return gth(codebook, idx)


def kernel(z, codebook):
    B, S_last, D = z.shape
    c2 = jnp.sum(codebook * codebook, axis=-1).reshape(1, _K)
    zt = jnp.transpose(z, (1, 0, 2))
    f_hat = None
    q = None
    for k, s in enumerate(_SCALES):
        s_prev = _SCALES[k - 1] if k else None
        if s < 8:
            if k == 0:
                r3 = _down_first(zt, s)
            else:
                f_hat, r3 = _glue(zt, f_hat, q, s_prev, s)
            idx = _nn_idx(r3.reshape(s * B, _D), codebook, c2)
            q = _sc_gather(codebook, idx, s * B).reshape(s, B, _D)
        else:
            grouped = s > 8
            f_hat, idx = _merged(zt, f_hat, q, codebook, c2, s_prev, s,
                                 grouped)
            bbs = 512 // s
            q = _sc_gather(codebook, idx, s * B) \
                .reshape(B // bbs, s, 2, bbs // 2, _D)
    f_hat, loss = _final(zt, f_hat, q)
    return jnp.transpose(f_hat, (1, 0, 2)), loss, loss
